# baseline (speedup 1.0000x reference)
import jax
import jax.numpy as jnp
from jax import lax
from jax.experimental import pallas as pl
from jax.experimental.pallas import tpu as pltpu
from jax.experimental.pallas import tpu_sc as plsc

MAX_POS = 8
PER_DIGIT_DIM = 64
BATCH = 16384

_NPAIR = MAX_POS // 2
_PAIR_DIM = 2 * PER_DIGIT_DIM
_NC = 2
_NS = 16
_NW = _NC * _NS
_B_PER_W = BATCH // _NW
_ROWS_PER_W = _B_PER_W * _NPAIR
_CHUNK = 256
_NBUF = 3
_NCHUNK = _ROWS_PER_W // _CHUNK


def _sc_body(pair_hbm, ns_hbm, out_hbm, ns_v, idx_v, rows_v, gsem, wsem):
    wid = lax.axis_index("s") * _NC + lax.axis_index("c")
    base_b = wid * _B_PER_W
    base_r = wid * _ROWS_PER_W

    pltpu.sync_copy(ns_hbm.at[pl.ds(base_b, _B_PER_W)], ns_v)

    lane4 = lax.iota(jnp.int32, 16) * _NPAIR

    def build(g, carry):
        q = ns_v[pl.ds(g * 16, 16)]
        base_idx = lane4 + g * (16 * _NPAIR)
        for i in range(_NPAIR):
            q2 = q // 100
            p = q - q2 * 100
            plsc.store_scatter(idx_v, [base_idx + i], p)
            q = q2
        return carry

    lax.fori_loop(0, _B_PER_W // 16, build, 0)

    def _gather_desc(c):
        return pltpu.make_async_copy(
            pair_hbm.at[idx_v.at[pl.ds(c * _CHUNK, _CHUNK)]],
            rows_v.at[c % _NBUF],
            gsem.at[c % _NBUF],
        )

    def _wb_desc(c):
        return pltpu.make_async_copy(
            rows_v.at[c % _NBUF],
            out_hbm.at[pl.ds(base_r + c * _CHUNK, _CHUNK)],
            wsem.at[c % _NBUF],
        )

    _gather_desc(0).start()
    _gather_desc(1).start()
    for c in range(_NCHUNK):
        _gather_desc(c).wait()
        _wb_desc(c).start()
        nc = c + 2
        if nc < _NCHUNK:
            if nc >= _NBUF:
                _wb_desc(nc - _NBUF).wait()
            _gather_desc(nc).start()
    for c in range(_NCHUNK - _NBUF, _NCHUNK):
        _wb_desc(c).wait()


@jax.jit
def _run(digit_bundles, ns):
    lo = jnp.broadcast_to(digit_bundles[None, :, :], (10, 10, PER_DIGIT_DIM))
    hi = jnp.broadcast_to(digit_bundles[:, None, :], (10, 10, PER_DIGIT_DIM))
    pair = jnp.concatenate(
        [lo.reshape(100, PER_DIGIT_DIM), hi.reshape(100, PER_DIGIT_DIM)], axis=-1
    )

    mesh = plsc.VectorSubcoreMesh(core_axis_name="c", subcore_axis_name="s")
    call = pl.kernel(
        _sc_body,
        out_type=jax.ShapeDtypeStruct((BATCH * _NPAIR, _PAIR_DIM), jnp.float32),
        mesh=mesh,
        scratch_types=[
            pltpu.VMEM((_B_PER_W,), jnp.int32),
            pltpu.VMEM((_ROWS_PER_W,), jnp.int32),
            pltpu.VMEM((_NBUF, _CHUNK, _PAIR_DIM), jnp.float32),
            pltpu.SemaphoreType.DMA((_NBUF,)),
            pltpu.SemaphoreType.DMA((_NBUF,)),
        ],
        compiler_params=pltpu.CompilerParams(needs_layout_passes=False),
    )
    rows = call(pair, ns)
    return rows.reshape(BATCH, MAX_POS * PER_DIGIT_DIM)


def kernel(digit_bundles, ns):
    return _run(digit_bundles, ns.astype(jnp.int32))


# speedup vs baseline: 1.9853x; 1.9853x over previous
import jax
import jax.numpy as jnp
from jax import lax
from jax.experimental import pallas as pl
from jax.experimental.pallas import tpu as pltpu
from jax.experimental.pallas import tpu_sc as plsc

MAX_POS = 8
PER_DIGIT_DIM = 64
BATCH = 16384

_NPAIR = MAX_POS // 2
_PAIR_DIM = 2 * PER_DIGIT_DIM
_NC = 2
_NS = 16
_NW = _NC * _NS
_B_PER_W = BATCH // _NW
_ROWS_PER_W = _B_PER_W * _NPAIR
_CHUNK = 256
_NBUF = 3
_NCHUNK = _ROWS_PER_W // _CHUNK


def _sc_body(pair_hbm, ns_hbm, out_hbm, pair_sh, ns_v, idx_v, rows_v, gsem, wsem):
    sid = lax.axis_index("s")
    wid = sid * _NC + lax.axis_index("c")
    base_b = wid * _B_PER_W
    base_r = wid * _ROWS_PER_W

    @pl.when(sid == 0)
    def _():
        pltpu.sync_copy(pair_hbm, pair_sh)

    pltpu.sync_copy(ns_hbm.at[pl.ds(base_b, _B_PER_W)], ns_v)
    plsc.subcore_barrier()

    lane4 = lax.iota(jnp.int32, 16) * _NPAIR

    def build(g, carry):
        q = ns_v[pl.ds(g * 16, 16)]
        base_idx = lane4 + g * (16 * _NPAIR)
        for i in range(_NPAIR):
            q2 = q // 100
            p = q - q2 * 100
            plsc.store_scatter(idx_v, [base_idx + i], p)
            q = q2
        return carry

    lax.fori_loop(0, _B_PER_W // 16, build, 0)

    def _gather_desc(c):
        return pltpu.make_async_copy(
            pair_sh.at[idx_v.at[pl.ds(c * _CHUNK, _CHUNK)]],
            rows_v.at[c % _NBUF],
            gsem.at[c % _NBUF],
        )

    def _wb_desc(c):
        return pltpu.make_async_copy(
            rows_v.at[c % _NBUF],
            out_hbm.at[pl.ds(base_r + c * _CHUNK, _CHUNK)],
            wsem.at[c % _NBUF],
        )

    _gather_desc(0).start()
    _gather_desc(1).start()
    for c in range(_NCHUNK):
        _gather_desc(c).wait()
        _wb_desc(c).start()
        nc = c + 2
        if nc < _NCHUNK:
            if nc >= _NBUF:
                _wb_desc(nc - _NBUF).wait()
            _gather_desc(nc).start()
    for c in range(_NCHUNK - _NBUF, _NCHUNK):
        _wb_desc(c).wait()


@jax.jit
def _run(digit_bundles, ns):
    lo = jnp.broadcast_to(digit_bundles[None, :, :], (10, 10, PER_DIGIT_DIM))
    hi = jnp.broadcast_to(digit_bundles[:, None, :], (10, 10, PER_DIGIT_DIM))
    pair = jnp.concatenate(
        [lo.reshape(100, PER_DIGIT_DIM), hi.reshape(100, PER_DIGIT_DIM)], axis=-1
    )

    mesh = plsc.VectorSubcoreMesh(core_axis_name="c", subcore_axis_name="s")
    call = pl.kernel(
        _sc_body,
        out_type=jax.ShapeDtypeStruct((BATCH * _NPAIR, _PAIR_DIM), jnp.float32),
        mesh=mesh,
        scratch_types=[
            pltpu.VMEM_SHARED((100, _PAIR_DIM), jnp.float32),
            pltpu.VMEM((_B_PER_W,), jnp.int32),
            pltpu.VMEM((_ROWS_PER_W,), jnp.int32),
            pltpu.VMEM((_NBUF, _CHUNK, _PAIR_DIM), jnp.float32),
            pltpu.SemaphoreType.DMA((_NBUF,)),
            pltpu.SemaphoreType.DMA((_NBUF,)),
        ],
        compiler_params=pltpu.CompilerParams(needs_layout_passes=False),
    )
    rows = call(pair, ns)
    return rows.reshape(BATCH, MAX_POS * PER_DIGIT_DIM)


def kernel(digit_bundles, ns):
    return _run(digit_bundles, ns.astype(jnp.int32))


# speedup vs baseline: 2.1228x; 1.0692x over previous
import jax
import jax.numpy as jnp
from jax import lax
from jax.experimental import pallas as pl
from jax.experimental.pallas import tpu as pltpu
from jax.experimental.pallas import tpu_sc as plsc

MAX_POS = 8
PER_DIGIT_DIM = 64
BATCH = 16384

_NPAIR = MAX_POS // 2
_PAIR_DIM = 2 * PER_DIGIT_DIM
_NC = 2
_NS = 16
_NW = _NC * _NS
_B_PER_W = BATCH // _NW
_ROWS_PER_W = _B_PER_W * _NPAIR
_CHUNK = 256
_NBUF = 3
_NCHUNK = _ROWS_PER_W // _CHUNK


def _sc_body(pair_hbm, ns_hbm, out_hbm, pair_sh, ns_v, idx_v, rows_v, gsem, wsem):
    sid = lax.axis_index("s")
    wid = sid * _NC + lax.axis_index("c")
    base_b = wid * _B_PER_W
    base_r = wid * _ROWS_PER_W

    @pl.when(sid == 0)
    def _():
        pltpu.sync_copy(pair_hbm, pair_sh)

    pltpu.sync_copy(ns_hbm.at[pl.ds(base_b, _B_PER_W)], ns_v)

    lane4 = lax.iota(jnp.int32, 16) * _NPAIR
    _G_PER_CHUNK = _CHUNK // (16 * _NPAIR)

    one = jnp.full((16,), 1, jnp.int32)
    zero = jnp.full((16,), 0, jnp.int32)

    def divmod100(q):
        t = (q.astype(jnp.float32) * 0.01).astype(jnp.int32)
        r = q - t * 100
        t = t + jnp.where(r >= 100, one, zero) - jnp.where(r < 0, one, zero)
        return t, q - t * 100

    def build_chunk(c):
        for g in range(c * _G_PER_CHUNK, (c + 1) * _G_PER_CHUNK):
            q = ns_v[pl.ds(g * 16, 16)]
            base_idx = lane4 + g * (16 * _NPAIR)
            for i in range(_NPAIR - 1):
                q, p = divmod100(q)
                plsc.store_scatter(idx_v, [base_idx + i], p)
            plsc.store_scatter(idx_v, [base_idx + _NPAIR - 1], q)

    def _gather_desc(c):
        return pltpu.make_async_copy(
            pair_sh.at[idx_v.at[pl.ds(c * _CHUNK, _CHUNK)]],
            rows_v.at[c % _NBUF],
            gsem.at[c % _NBUF],
        )

    def _wb_desc(c):
        return pltpu.make_async_copy(
            rows_v.at[c % _NBUF],
            out_hbm.at[pl.ds(base_r + c * _CHUNK, _CHUNK)],
            wsem.at[c % _NBUF],
        )

    build_chunk(0)
    plsc.subcore_barrier()
    _gather_desc(0).start()
    build_chunk(1)
    _gather_desc(1).start()
    for c in range(_NCHUNK):
        nc = c + 2
        if nc < _NCHUNK:
            build_chunk(nc)
            if nc >= _NBUF:
                _wb_desc(nc - _NBUF).wait()
            _gather_desc(nc).start()
        _gather_desc(c).wait()
        _wb_desc(c).start()
    for c in range(_NCHUNK - _NBUF, _NCHUNK):
        _wb_desc(c).wait()


@jax.jit
def _run(digit_bundles, ns):
    lo = jnp.broadcast_to(digit_bundles[None, :, :], (10, 10, PER_DIGIT_DIM))
    hi = jnp.broadcast_to(digit_bundles[:, None, :], (10, 10, PER_DIGIT_DIM))
    pair = jnp.concatenate(
        [lo.reshape(100, PER_DIGIT_DIM), hi.reshape(100, PER_DIGIT_DIM)], axis=-1
    )

    mesh = plsc.VectorSubcoreMesh(core_axis_name="c", subcore_axis_name="s")
    call = pl.kernel(
        _sc_body,
        out_type=jax.ShapeDtypeStruct((BATCH * _NPAIR, _PAIR_DIM), jnp.float32),
        mesh=mesh,
        scratch_types=[
            pltpu.VMEM_SHARED((100, _PAIR_DIM), jnp.float32),
            pltpu.VMEM((_B_PER_W,), jnp.int32),
            pltpu.VMEM((_ROWS_PER_W,), jnp.int32),
            pltpu.VMEM((_NBUF, _CHUNK, _PAIR_DIM), jnp.float32),
            pltpu.SemaphoreType.DMA((_NBUF,)),
            pltpu.SemaphoreType.DMA((_NBUF,)),
        ],
        compiler_params=pltpu.CompilerParams(needs_layout_passes=False),
    )
    rows = call(pair, ns)
    return rows.reshape(BATCH, MAX_POS * PER_DIGIT_DIM)


def kernel(digit_bundles, ns):
    return _run(digit_bundles, ns.astype(jnp.int32))


# speedup vs baseline: 2.1497x; 1.0127x over previous
import jax
import jax.numpy as jnp
from jax import lax
from jax.experimental import pallas as pl
from jax.experimental.pallas import tpu as pltpu
from jax.experimental.pallas import tpu_sc as plsc

MAX_POS = 8
PER_DIGIT_DIM = 64
BATCH = 16384

_NPAIR = MAX_POS // 2
_PAIR_DIM = 2 * PER_DIGIT_DIM
_NC = 2
_NS = 16
_NW = _NC * _NS
_B_PER_W = BATCH // _NW
_ROWS_PER_W = _B_PER_W * _NPAIR
_CHUNK = 128
_NBUF = 6
_NCHUNK = _ROWS_PER_W // _CHUNK
_PRIME = 4


def _sc_body(pair_hbm, ns_hbm, out_hbm, pair_sh, ns_v, idx_v, rows_v, gsem, wsem):
    sid = lax.axis_index("s")
    wid = sid * _NC + lax.axis_index("c")
    base_b = wid * _B_PER_W
    base_r = wid * _ROWS_PER_W

    @pl.when(sid == 0)
    def _():
        pltpu.sync_copy(pair_hbm, pair_sh)

    pltpu.sync_copy(ns_hbm.at[pl.ds(base_b, _B_PER_W)], ns_v)

    lane4 = lax.iota(jnp.int32, 16) * _NPAIR
    _G_PER_CHUNK = _CHUNK // (16 * _NPAIR)

    one = jnp.full((16,), 1, jnp.int32)
    zero = jnp.full((16,), 0, jnp.int32)

    def divmod100(q):
        t = (q.astype(jnp.float32) * 0.01).astype(jnp.int32)
        r = q - t * 100
        t = t + jnp.where(r >= 100, one, zero) - jnp.where(r < 0, one, zero)
        return t, q - t * 100

    def build_chunk(c):
        for g in range(c * _G_PER_CHUNK, (c + 1) * _G_PER_CHUNK):
            q = ns_v[pl.ds(g * 16, 16)]
            base_idx = lane4 + g * (16 * _NPAIR)
            for i in range(_NPAIR - 1):
                q, p = divmod100(q)
                plsc.store_scatter(idx_v, [base_idx + i], p)
            plsc.store_scatter(idx_v, [base_idx + _NPAIR - 1], q)

    def _gather_desc(c):
        return pltpu.make_async_copy(
            pair_sh.at[idx_v.at[pl.ds(c * _CHUNK, _CHUNK)]],
            rows_v.at[c % _NBUF],
            gsem.at[c % _NBUF],
        )

    def _wb_desc(c):
        return pltpu.make_async_copy(
            rows_v.at[c % _NBUF],
            out_hbm.at[pl.ds(base_r + c * _CHUNK, _CHUNK)],
            wsem.at[c % _NBUF],
        )

    build_chunk(0)
    plsc.subcore_barrier()
    _gather_desc(0).start()
    for k in range(1, _PRIME):
        build_chunk(k)
        _gather_desc(k).start()
    for c in range(_NCHUNK):
        nc = c + _PRIME
        if nc < _NCHUNK:
            build_chunk(nc)
            if nc >= _NBUF:
                _wb_desc(nc - _NBUF).wait()
            _gather_desc(nc).start()
        _gather_desc(c).wait()
        _wb_desc(c).start()
    for c in range(_NCHUNK - _NBUF, _NCHUNK):
        _wb_desc(c).wait()


@jax.jit
def _run(digit_bundles, ns):
    lo = jnp.broadcast_to(digit_bundles[None, :, :], (10, 10, PER_DIGIT_DIM))
    hi = jnp.broadcast_to(digit_bundles[:, None, :], (10, 10, PER_DIGIT_DIM))
    pair = jnp.concatenate(
        [lo.reshape(100, PER_DIGIT_DIM), hi.reshape(100, PER_DIGIT_DIM)], axis=-1
    )

    mesh = plsc.VectorSubcoreMesh(core_axis_name="c", subcore_axis_name="s")
    call = pl.kernel(
        _sc_body,
        out_type=jax.ShapeDtypeStruct((BATCH * _NPAIR, _PAIR_DIM), jnp.float32),
        mesh=mesh,
        scratch_types=[
            pltpu.VMEM_SHARED((100, _PAIR_DIM), jnp.float32),
            pltpu.VMEM((_B_PER_W,), jnp.int32),
            pltpu.VMEM((_ROWS_PER_W,), jnp.int32),
            pltpu.VMEM((_NBUF, _CHUNK, _PAIR_DIM), jnp.float32),
            pltpu.SemaphoreType.DMA((_NBUF,)),
            pltpu.SemaphoreType.DMA((_NBUF,)),
        ],
        compiler_params=pltpu.CompilerParams(needs_layout_passes=False),
    )
    rows = call(pair, ns)
    return rows.reshape(BATCH, MAX_POS * PER_DIGIT_DIM)


def kernel(digit_bundles, ns):
    return _run(digit_bundles, ns.astype(jnp.int32))
